# baseline (speedup 1.0000x reference)
import functools

import numpy as np
import jax
import jax.numpy as jnp
from jax.experimental import pallas as pl
from jax.experimental.pallas import tpu as pltpu


def _interp_matrix_np(out_size, in_size):
    if in_size == 1:
        return np.ones((out_size, 1), np.float32)
    if out_size == 1:
        src = np.zeros((1,), np.float64)
    else:
        src = np.arange(out_size, dtype=np.float64) * (in_size - 1) / (out_size - 1)
    i0 = np.clip(np.floor(src).astype(np.int64), 0, in_size - 1)
    i1 = np.minimum(i0 + 1, in_size - 1)
    frac = (src - i0).astype(np.float32)
    R = np.zeros((out_size, in_size), np.float32)
    R[np.arange(out_size), i0] += 1.0 - frac
    R[np.arange(out_size), i1] += frac
    return R


def _mega_kernel(x_ref, sall_ref, wselb_ref, poolwt_ref, poolbt_ref,
                 wmt_ref, bwpt_ref, bbt_ref, cwt_ref, cbt_ref, rh_ref, rwt_ref,
                 o_ref, selall_ref, selcat_ref, ft_ref, feats_ref, lg_ref,
                 v3_ref, *, dims, inv_p):
    C, h, w, H, W = dims
    C64 = C * 64
    t = pl.program_id(1)

    @pl.when(t == 0)
    def _head():
        E = ft_ref.shape[0]
        K = cwt_ref.shape[0]
        P = h * w

        x2 = x_ref[0].reshape(C * H, W)
        selall_ref[...] = jnp.dot(x2, sall_ref[...],
                                  preferred_element_type=jnp.float32)

        selcat_ref[pl.ds(C64, 8), :] = jnp.ones((8, P), jnp.float32)

        def band(i4, _):
            for c in range(C):
                b = selall_ref[pl.ds(c * H + 32 * i4, 32), :]
                for dx in range(8):
                    piece = jnp.concatenate(
                        [b[8 * s:8 * (s + 1), w * dx:w * (dx + 1)]
                         for s in range(4)], axis=1)
                    selcat_ref[pl.ds(dx * C * 8 + c * 8, 8),
                               pl.ds(4 * w * i4, 4 * w)] = piece
            return 0

        jax.lax.fori_loop(0, h // 4, band, 0)

        f = jnp.maximum(jnp.dot(wselb_ref[...], selcat_ref[...],
                                preferred_element_type=jnp.float32), 0.0)
        ft_ref[...] = f.astype(jnp.bfloat16)

        mean = jnp.sum(f, axis=1, keepdims=True) * inv_p
        hdn = jnp.dot(poolwt_ref[...], mean,
                      preferred_element_type=jnp.float32) + poolbt_ref[...]
        hdn = jnp.maximum(hdn, 0.0)
        pc = jnp.dot(bwpt_ref[...], hdn,
                     preferred_element_type=jnp.float32) + bbt_ref[...]

        feats = jnp.dot(wmt_ref[...], ft_ref[...],
                        preferred_element_type=jnp.float32)
        feats_ref[...] = jnp.maximum(feats + pc, 0.0).astype(jnp.bfloat16)
        lg_ref[...] = jnp.dot(cwt_ref[...], feats_ref[...],
                              preferred_element_type=jnp.float32) + cbt_ref[...]

        def colpass(i4, _):
            lgp = lg_ref[:, pl.ds(4 * w * i4, 4 * w)]
            for s in range(4):
                v = jnp.dot(lgp[:, w * s:w * (s + 1)].astype(jnp.bfloat16),
                            rwt_ref[...], preferred_element_type=jnp.float32)
                v3_ref[pl.ds(4 * i4 + s, 1)] = (
                    v.astype(jnp.bfloat16).reshape(1, K, W))
            return 0

        jax.lax.fori_loop(0, h // 4, colpass, 0)

    K = cwt_ref.shape[0]
    for kk in range(K):
        vk = v3_ref[:, pl.ds(kk, 1), :].reshape(h, W)
        o_ref[0, kk] = jnp.dot(rh_ref[...], vk,
                               preferred_element_type=jnp.float32)


def kernel(patch_w, patch_b, pool_w, pool_b, bott_w_main, bott_w_pool,
           bott_b, cls_w, cls_b, x):
    N, C, H, W = x.shape
    h, w = H // 8, W // 8
    P = h * w
    C64 = C * 64
    E = patch_w.shape[1]
    Pd = pool_w.shape[1]
    Bn = bott_w_main.shape[1]
    K = cls_w.shape[1]
    T8 = 8
    Ht = H // T8

    sall_np = np.zeros((W, W), np.float32)
    for dx in range(8):
        sall_np[8 * np.arange(w) + dx, dx * w + np.arange(w)] = 1.0
    sall = jnp.asarray(sall_np)

    wsel = (patch_w.reshape(C, 8, 8, E).transpose(2, 0, 1, 3)
            .reshape(C64, E).T)
    wselb = jnp.concatenate(
        [wsel.astype(jnp.float32), patch_b.T,
         jnp.zeros((E, 7), jnp.float32)], axis=1)

    poolwt = pool_w.T
    poolbt = pool_b.T
    wmt = bott_w_main.T
    bwpt = bott_w_pool.T
    bbt = bott_b.T
    cwt = cls_w.T
    cbt = cls_b.T

    Rh = jnp.asarray(_interp_matrix_np(H, h), jnp.bfloat16)
    Rw_T = jnp.asarray(_interp_matrix_np(W, w).T,
                       jnp.bfloat16)

    return pl.pallas_call(
        functools.partial(_mega_kernel, dims=(C, h, w, H, W),
                          inv_p=1.0 / float(P)),
        grid=(N, T8),
        in_specs=[
            pl.BlockSpec((1, C, H, W), lambda n, t: (n, 0, 0, 0)),
            pl.BlockSpec((W, W), lambda n, t: (0, 0)),
            pl.BlockSpec((E, C64 + 8), lambda n, t: (0, 0)),
            pl.BlockSpec((Pd, E), lambda n, t: (0, 0)),
            pl.BlockSpec((Pd, 1), lambda n, t: (0, 0)),
            pl.BlockSpec((Bn, E), lambda n, t: (0, 0)),
            pl.BlockSpec((Bn, Pd), lambda n, t: (0, 0)),
            pl.BlockSpec((Bn, 1), lambda n, t: (0, 0)),
            pl.BlockSpec((K, Bn), lambda n, t: (0, 0)),
            pl.BlockSpec((K, 1), lambda n, t: (0, 0)),
            pl.BlockSpec((Ht, h), lambda n, t: (t, 0)),
            pl.BlockSpec((w, W), lambda n, t: (0, 0)),
        ],
        out_specs=pl.BlockSpec((1, K, Ht, W), lambda n, t: (n, 0, t, 0)),
        out_shape=jax.ShapeDtypeStruct((N, K, H, W), jnp.float32),
        scratch_shapes=[
            pltpu.VMEM((C * H, W), jnp.float32),
            pltpu.VMEM((C64 + 8, P), jnp.float32),
            pltpu.VMEM((E, P), jnp.bfloat16),
            pltpu.VMEM((Bn, P), jnp.bfloat16),
            pltpu.VMEM((K, P), jnp.float32),
            pltpu.VMEM((h, K, W), jnp.bfloat16),
        ],
        compiler_params=pltpu.CompilerParams(
            dimension_semantics=("parallel", "arbitrary")),
    )(x, sall, wselb, poolwt, poolbt, wmt, bwpt, bbt, cwt, cbt, Rh, Rw_T)


# speedup vs baseline: 1.3414x; 1.3414x over previous
import functools

import numpy as np
import jax
import jax.numpy as jnp
from jax.experimental import pallas as pl
from jax.experimental.pallas import tpu as pltpu


def _interp_matrix_np(out_size, in_size):
    if in_size == 1:
        return np.ones((out_size, 1), np.float32)
    if out_size == 1:
        src = np.zeros((1,), np.float64)
    else:
        src = np.arange(out_size, dtype=np.float64) * (in_size - 1) / (out_size - 1)
    i0 = np.clip(np.floor(src).astype(np.int64), 0, in_size - 1)
    i1 = np.minimum(i0 + 1, in_size - 1)
    frac = (src - i0).astype(np.float32)
    R = np.zeros((out_size, in_size), np.float32)
    R[np.arange(out_size), i0] += 1.0 - frac
    R[np.arange(out_size), i1] += frac
    return R


def _mega_kernel(x_ref, sall_ref, wselb_ref, poolwt_ref, poolbt_ref,
                 wmt_ref, bwpt_ref, bbt_ref, cwt_ref, cbt_ref, rh_ref, rwt_ref,
                 o_ref, selall_ref, selcat_ref, ft_ref, feats_ref, lg_ref,
                 v3_ref, *, dims, inv_p):
    C, h, w, H, W = dims
    C64 = C * 64
    t = pl.program_id(1)

    @pl.when(t == 0)
    def _head():
        E = ft_ref.shape[0]
        K = cwt_ref.shape[0]
        P = h * w

        x2 = x_ref[0].reshape(C * H, W)
        selall_ref[...] = jnp.dot(x2, sall_ref[...],
                                  preferred_element_type=jnp.float32)

        selcat_ref[pl.ds(C64, 8), :] = jnp.ones((8, P), jnp.float32)

        def band(i4, _):
            for c in range(C):
                b = selall_ref[pl.ds(c * H + 32 * i4, 32), :]
                for dx in range(8):
                    piece = jnp.concatenate(
                        [b[8 * s:8 * (s + 1), w * dx:w * (dx + 1)]
                         for s in range(4)], axis=1)
                    selcat_ref[pl.ds(dx * C * 8 + c * 8, 8),
                               pl.ds(4 * w * i4, 4 * w)] = piece
            return 0

        jax.lax.fori_loop(0, h // 4, band, 0)

        f = jnp.maximum(jnp.dot(wselb_ref[...], selcat_ref[...],
                                preferred_element_type=jnp.float32), 0.0)
        ft_ref[...] = f.astype(jnp.bfloat16)

        mean = jnp.sum(f, axis=1, keepdims=True) * inv_p
        hdn = jnp.dot(poolwt_ref[...], mean,
                      preferred_element_type=jnp.float32) + poolbt_ref[...]
        hdn = jnp.maximum(hdn, 0.0)
        pc = jnp.dot(bwpt_ref[...], hdn,
                     preferred_element_type=jnp.float32) + bbt_ref[...]

        feats = jnp.dot(wmt_ref[...], ft_ref[...],
                        preferred_element_type=jnp.float32)
        feats_ref[...] = jnp.maximum(feats + pc, 0.0).astype(jnp.bfloat16)
        lg_ref[...] = jnp.dot(cwt_ref[...], feats_ref[...],
                              preferred_element_type=jnp.float32) + cbt_ref[...]

        def colpass(i4, _):
            lgp = lg_ref[:, pl.ds(4 * w * i4, 4 * w)]
            for s in range(4):
                v = jnp.dot(lgp[:, w * s:w * (s + 1)].astype(jnp.bfloat16),
                            rwt_ref[...], preferred_element_type=jnp.float32)
                v3_ref[pl.ds(4 * i4 + s, 1)] = v.reshape(1, K, W)
            return 0

        jax.lax.fori_loop(0, h // 4, colpass, 0)

    vk = v3_ref[:, pl.ds(t, 1), :].reshape(h, W)
    o_ref[0, 0] = jnp.dot(rh_ref[...], vk, preferred_element_type=jnp.float32)


def kernel(patch_w, patch_b, pool_w, pool_b, bott_w_main, bott_w_pool,
           bott_b, cls_w, cls_b, x):
    N, C, H, W = x.shape
    h, w = H // 8, W // 8
    P = h * w
    C64 = C * 64
    E = patch_w.shape[1]
    Pd = pool_w.shape[1]
    Bn = bott_w_main.shape[1]
    K = cls_w.shape[1]

    sall_np = np.zeros((W, W), np.float32)
    for dx in range(8):
        sall_np[8 * np.arange(w) + dx, dx * w + np.arange(w)] = 1.0
    sall = jnp.asarray(sall_np)

    wsel = (patch_w.reshape(C, 8, 8, E).transpose(2, 0, 1, 3)
            .reshape(C64, E).T)
    wselb = jnp.concatenate(
        [wsel.astype(jnp.float32), patch_b.T,
         jnp.zeros((E, 7), jnp.float32)], axis=1)

    poolwt = pool_w.T
    poolbt = pool_b.T
    wmt = bott_w_main.T
    bwpt = bott_w_pool.T
    bbt = bott_b.T
    cwt = cls_w.T
    cbt = cls_b.T

    Rh = jnp.asarray(_interp_matrix_np(H, h))
    Rw_T = jnp.asarray(_interp_matrix_np(W, w).T,
                       jnp.bfloat16)

    return pl.pallas_call(
        functools.partial(_mega_kernel, dims=(C, h, w, H, W),
                          inv_p=1.0 / float(P)),
        grid=(N, K),
        in_specs=[
            pl.BlockSpec((1, C, H, W), lambda n, t: (n, 0, 0, 0)),
            pl.BlockSpec((W, W), lambda n, t: (0, 0)),
            pl.BlockSpec((E, C64 + 8), lambda n, t: (0, 0)),
            pl.BlockSpec((Pd, E), lambda n, t: (0, 0)),
            pl.BlockSpec((Pd, 1), lambda n, t: (0, 0)),
            pl.BlockSpec((Bn, E), lambda n, t: (0, 0)),
            pl.BlockSpec((Bn, Pd), lambda n, t: (0, 0)),
            pl.BlockSpec((Bn, 1), lambda n, t: (0, 0)),
            pl.BlockSpec((K, Bn), lambda n, t: (0, 0)),
            pl.BlockSpec((K, 1), lambda n, t: (0, 0)),
            pl.BlockSpec((H, h), lambda n, t: (0, 0)),
            pl.BlockSpec((w, W), lambda n, t: (0, 0)),
        ],
        out_specs=pl.BlockSpec((1, 1, H, W), lambda n, t: (n, t, 0, 0)),
        out_shape=jax.ShapeDtypeStruct((N, K, H, W), jnp.float32),
        scratch_shapes=[
            pltpu.VMEM((C * H, W), jnp.float32),
            pltpu.VMEM((C64 + 8, P), jnp.float32),
            pltpu.VMEM((E, P), jnp.bfloat16),
            pltpu.VMEM((Bn, P), jnp.bfloat16),
            pltpu.VMEM((K, P), jnp.float32),
            pltpu.VMEM((h, K, W), jnp.float32),
        ],
        compiler_params=pltpu.CompilerParams(
            dimension_semantics=("parallel", "arbitrary")),
    )(x, sall, wselb, poolwt, poolbt, wmt, bwpt, bbt, cwt, cbt, Rh, Rw_T)


# speedup vs baseline: 1.5952x; 1.1892x over previous
import functools

import numpy as np
import jax
import jax.numpy as jnp
from jax.experimental import pallas as pl
from jax.experimental.pallas import tpu as pltpu


def _interp_matrix_np(out_size, in_size):
    if in_size == 1:
        return np.ones((out_size, 1), np.float32)
    if out_size == 1:
        src = np.zeros((1,), np.float64)
    else:
        src = np.arange(out_size, dtype=np.float64) * (in_size - 1) / (out_size - 1)
    i0 = np.clip(np.floor(src).astype(np.int64), 0, in_size - 1)
    i1 = np.minimum(i0 + 1, in_size - 1)
    frac = (src - i0).astype(np.float32)
    R = np.zeros((out_size, in_size), np.float32)
    R[np.arange(out_size), i0] += 1.0 - frac
    R[np.arange(out_size), i1] += frac
    return R


def _mega_kernel(x_ref, sall_ref, wselb_ref, poolwt_ref, poolbt_ref,
                 wmt_ref, bwpt_ref, bbt_ref, cwt_ref, cbt_ref, rh_ref, rwt_ref,
                 o_ref, selall_ref, selcat_ref, ft_ref, feats_ref, lg_ref,
                 v3_ref, obuf_ref, osem, *, dims, inv_p, nk):
    C, h, w, H, W = dims
    C64 = C * 64
    n = pl.program_id(0)
    t = pl.program_id(1)
    K = cwt_ref.shape[0]
    D = obuf_ref.shape[0]
    sid = n * K + t
    slot = jax.lax.rem(sid, D)

    @pl.when(t == 0)
    def _head():
        E = ft_ref.shape[0]
        K = cwt_ref.shape[0]
        P = h * w

        x2 = x_ref[0].reshape(C * H, W)
        selall_ref[...] = jnp.dot(x2, sall_ref[...],
                                  preferred_element_type=jnp.float32)

        selcat_ref[pl.ds(C64, 8), :] = jnp.ones((8, P), jnp.float32)

        def band(i4, _):
            for c in range(C):
                b = selall_ref[pl.ds(c * H + 32 * i4, 32), :]
                for dx in range(8):
                    piece = jnp.concatenate(
                        [b[8 * s:8 * (s + 1), w * dx:w * (dx + 1)]
                         for s in range(4)], axis=1)
                    selcat_ref[pl.ds(dx * C * 8 + c * 8, 8),
                               pl.ds(4 * w * i4, 4 * w)] = piece
            return 0

        jax.lax.fori_loop(0, h // 4, band, 0)

        f = jnp.maximum(jnp.dot(wselb_ref[...], selcat_ref[...],
                                preferred_element_type=jnp.float32), 0.0)
        ft_ref[...] = f.astype(jnp.bfloat16)

        mean = jnp.sum(f, axis=1, keepdims=True) * inv_p
        hdn = jnp.dot(poolwt_ref[...], mean,
                      preferred_element_type=jnp.float32) + poolbt_ref[...]
        hdn = jnp.maximum(hdn, 0.0)
        pc = jnp.dot(bwpt_ref[...], hdn,
                     preferred_element_type=jnp.float32) + bbt_ref[...]

        feats = jnp.dot(wmt_ref[...], ft_ref[...],
                        preferred_element_type=jnp.float32)
        feats_ref[...] = jnp.maximum(feats + pc, 0.0).astype(jnp.bfloat16)
        lg_ref[...] = jnp.dot(cwt_ref[...], feats_ref[...],
                              preferred_element_type=jnp.float32) + cbt_ref[...]

        def colpass(i4, _):
            lgp = lg_ref[:, pl.ds(4 * w * i4, 4 * w)]
            for s in range(4):
                v = jnp.dot(lgp[:, w * s:w * (s + 1)].astype(jnp.bfloat16),
                            rwt_ref[...], preferred_element_type=jnp.float32)
                v3_ref[pl.ds(4 * i4 + s, 1)] = v.reshape(1, K, W)
            return 0

        jax.lax.fori_loop(0, h // 4, colpass, 0)

    @pl.when(sid >= D)
    def _reclaim():
        pltpu.make_async_copy(obuf_ref.at[slot], o_ref.at[0, 0],
                              osem.at[slot]).wait()

    vk = v3_ref[:, pl.ds(t, 1), :].reshape(h, W)
    obuf_ref[slot] = jnp.dot(rh_ref[...], vk,
                             preferred_element_type=jnp.float32)
    pltpu.make_async_copy(obuf_ref.at[slot], o_ref.at[n, t],
                          osem.at[slot]).start()

    @pl.when(sid == nk - 1)
    def _drain():
        for d in range(D):
            pltpu.make_async_copy(obuf_ref.at[d], o_ref.at[0, 0],
                                  osem.at[d]).wait()


def kernel(patch_w, patch_b, pool_w, pool_b, bott_w_main, bott_w_pool,
           bott_b, cls_w, cls_b, x):
    N, C, H, W = x.shape
    h, w = H // 8, W // 8
    P = h * w
    C64 = C * 64
    E = patch_w.shape[1]
    Pd = pool_w.shape[1]
    Bn = bott_w_main.shape[1]
    K = cls_w.shape[1]

    sall_np = np.zeros((W, W), np.float32)
    for dx in range(8):
        sall_np[8 * np.arange(w) + dx, dx * w + np.arange(w)] = 1.0
    sall = jnp.asarray(sall_np)

    wsel = (patch_w.reshape(C, 8, 8, E).transpose(2, 0, 1, 3)
            .reshape(C64, E).T)
    wselb = jnp.concatenate(
        [wsel.astype(jnp.float32), patch_b.T,
         jnp.zeros((E, 7), jnp.float32)], axis=1)

    poolwt = pool_w.T
    poolbt = pool_b.T
    wmt = bott_w_main.T
    bwpt = bott_w_pool.T
    bbt = bott_b.T
    cwt = cls_w.T
    cbt = cls_b.T

    Rh = jnp.asarray(_interp_matrix_np(H, h))
    Rw_T = jnp.asarray(_interp_matrix_np(W, w).T,
                       jnp.bfloat16)

    return pl.pallas_call(
        functools.partial(_mega_kernel, dims=(C, h, w, H, W),
                          inv_p=1.0 / float(P), nk=N * K),
        grid=(N, K),
        in_specs=[
            pl.BlockSpec((1, C, H, W), lambda n, t: (n, 0, 0, 0)),
            pl.BlockSpec((W, W), lambda n, t: (0, 0)),
            pl.BlockSpec((E, C64 + 8), lambda n, t: (0, 0)),
            pl.BlockSpec((Pd, E), lambda n, t: (0, 0)),
            pl.BlockSpec((Pd, 1), lambda n, t: (0, 0)),
            pl.BlockSpec((Bn, E), lambda n, t: (0, 0)),
            pl.BlockSpec((Bn, Pd), lambda n, t: (0, 0)),
            pl.BlockSpec((Bn, 1), lambda n, t: (0, 0)),
            pl.BlockSpec((K, Bn), lambda n, t: (0, 0)),
            pl.BlockSpec((K, 1), lambda n, t: (0, 0)),
            pl.BlockSpec((H, h), lambda n, t: (0, 0)),
            pl.BlockSpec((w, W), lambda n, t: (0, 0)),
        ],
        out_specs=pl.BlockSpec(memory_space=pltpu.MemorySpace.HBM),
        out_shape=jax.ShapeDtypeStruct((N, K, H, W), jnp.float32),
        scratch_shapes=[
            pltpu.VMEM((C * H, W), jnp.float32),
            pltpu.VMEM((C64 + 8, P), jnp.float32),
            pltpu.VMEM((E, P), jnp.bfloat16),
            pltpu.VMEM((Bn, P), jnp.bfloat16),
            pltpu.VMEM((K, P), jnp.float32),
            pltpu.VMEM((h, K, W), jnp.float32),
            pltpu.VMEM((12, H, W), jnp.float32),
            pltpu.SemaphoreType.DMA((12,)),
        ],
        compiler_params=pltpu.CompilerParams(
            dimension_semantics=("arbitrary", "arbitrary")),
    )(x, sall, wselb, poolwt, poolbt, wmt, bwpt, bbt, cwt, cbt, Rh, Rw_T)


# speedup vs baseline: 1.6982x; 1.0646x over previous
import functools

import numpy as np
import jax
import jax.numpy as jnp
from jax.experimental import pallas as pl
from jax.experimental.pallas import tpu as pltpu


def _interp_matrix_np(out_size, in_size):
    if in_size == 1:
        return np.ones((out_size, 1), np.float32)
    if out_size == 1:
        src = np.zeros((1,), np.float64)
    else:
        src = np.arange(out_size, dtype=np.float64) * (in_size - 1) / (out_size - 1)
    i0 = np.clip(np.floor(src).astype(np.int64), 0, in_size - 1)
    i1 = np.minimum(i0 + 1, in_size - 1)
    frac = (src - i0).astype(np.float32)
    R = np.zeros((out_size, in_size), np.float32)
    R[np.arange(out_size), i0] += 1.0 - frac
    R[np.arange(out_size), i1] += frac
    return R


def _mega_kernel(x_ref, sall_ref, wselb_ref, poolwt_ref, poolbt_ref,
                 wmt_ref, bwpt_ref, bbt_ref, cwt_ref, cbt_ref, rh_ref, rwt_ref,
                 o_ref, selall_ref, selcat_ref, ft_ref, feats_ref, lg_ref,
                 v3_ref, obuf_ref, osem, *, dims, inv_p, nk):
    C, h, w, H, W = dims
    C64 = C * 64
    n = pl.program_id(0)
    t = pl.program_id(1)
    K = cwt_ref.shape[0]
    D = obuf_ref.shape[0]
    sid = n * K + t
    slot = jax.lax.rem(sid, D)

    @pl.when(t == 0)
    def _head():
        E = ft_ref.shape[0]
        K = cwt_ref.shape[0]
        P = h * w

        x2 = x_ref[0].reshape(C * H, W)
        selall_ref[...] = jnp.dot(x2, sall_ref[...],
                                  preferred_element_type=jnp.float32)

        selcat_ref[pl.ds(C64, 8), :] = jnp.ones((8, P), jnp.float32)

        def band(i4, _):
            for c in range(C):
                b = selall_ref[pl.ds(c * H + 32 * i4, 32), :]
                for dx in range(8):
                    piece = jnp.concatenate(
                        [b[8 * s:8 * (s + 1), w * dx:w * (dx + 1)]
                         for s in range(4)], axis=1)
                    selcat_ref[pl.ds(dx * C * 8 + c * 8, 8),
                               pl.ds(4 * w * i4, 4 * w)] = piece
            return 0

        jax.lax.fori_loop(0, h // 4, band, 0)

        f = jnp.maximum(jnp.dot(wselb_ref[...], selcat_ref[...],
                                preferred_element_type=jnp.float32), 0.0)
        ft_ref[...] = f.astype(jnp.bfloat16)

        mean = jnp.sum(f, axis=1, keepdims=True) * inv_p
        hdn = jnp.dot(poolwt_ref[...], mean,
                      preferred_element_type=jnp.float32) + poolbt_ref[...]
        hdn = jnp.maximum(hdn, 0.0)
        pc = jnp.dot(bwpt_ref[...], hdn,
                     preferred_element_type=jnp.float32) + bbt_ref[...]

        feats = jnp.dot(wmt_ref[...], ft_ref[...],
                        preferred_element_type=jnp.float32)
        feats_ref[...] = jnp.maximum(feats + pc, 0.0).astype(jnp.bfloat16)
        lg_ref[...] = (jnp.dot(cwt_ref[...], feats_ref[...],
                               preferred_element_type=jnp.float32)
                       + cbt_ref[...]).astype(jnp.bfloat16)

        def colpass(i16, _):
            lgp = lg_ref[:, pl.ds(16 * w * i16, 16 * w)]
            vs = [jnp.dot(lgp[:, w * s:w * (s + 1)], rwt_ref[...],
                          preferred_element_type=jnp.float32)
                  .astype(jnp.bfloat16).reshape(K, 1, W)
                  for s in range(16)]
            v3_ref[:, pl.ds(16 * i16, 16), :] = jnp.concatenate(vs, axis=1)
            return 0

        jax.lax.fori_loop(0, h // 16, colpass, 0)

    @pl.when(sid >= D)
    def _reclaim():
        pltpu.make_async_copy(obuf_ref.at[slot], o_ref.at[0, 0],
                              osem.at[slot]).wait()

    vk = v3_ref[t]
    obuf_ref[slot] = jnp.dot(rh_ref[...], vk,
                             preferred_element_type=jnp.float32)
    pltpu.make_async_copy(obuf_ref.at[slot], o_ref.at[n, t],
                          osem.at[slot]).start()

    @pl.when(sid == nk - 1)
    def _drain():
        for d in range(D):
            pltpu.make_async_copy(obuf_ref.at[d], o_ref.at[0, 0],
                                  osem.at[d]).wait()


def kernel(patch_w, patch_b, pool_w, pool_b, bott_w_main, bott_w_pool,
           bott_b, cls_w, cls_b, x):
    N, C, H, W = x.shape
    h, w = H // 8, W // 8
    P = h * w
    C64 = C * 64
    E = patch_w.shape[1]
    Pd = pool_w.shape[1]
    Bn = bott_w_main.shape[1]
    K = cls_w.shape[1]

    sall_np = np.zeros((W, W), np.float32)
    for dx in range(8):
        sall_np[8 * np.arange(w) + dx, dx * w + np.arange(w)] = 1.0
    sall = jnp.asarray(sall_np)

    wsel = (patch_w.reshape(C, 8, 8, E).transpose(2, 0, 1, 3)
            .reshape(C64, E).T)
    wselb = jnp.concatenate(
        [wsel.astype(jnp.float32), patch_b.T,
         jnp.zeros((E, 7), jnp.float32)], axis=1)

    poolwt = pool_w.T
    poolbt = pool_b.T
    wmt = bott_w_main.T
    bwpt = bott_w_pool.T
    bbt = bott_b.T
    cwt = cls_w.T
    cbt = cls_b.T

    Rh = jnp.asarray(_interp_matrix_np(H, h), jnp.bfloat16)
    Rw_T = jnp.asarray(_interp_matrix_np(W, w).T,
                       jnp.bfloat16)

    return pl.pallas_call(
        functools.partial(_mega_kernel, dims=(C, h, w, H, W),
                          inv_p=1.0 / float(P), nk=N * K),
        grid=(N, K),
        in_specs=[
            pl.BlockSpec((1, C, H, W), lambda n, t: (n, 0, 0, 0)),
            pl.BlockSpec((W, W), lambda n, t: (0, 0)),
            pl.BlockSpec((E, C64 + 8), lambda n, t: (0, 0)),
            pl.BlockSpec((Pd, E), lambda n, t: (0, 0)),
            pl.BlockSpec((Pd, 1), lambda n, t: (0, 0)),
            pl.BlockSpec((Bn, E), lambda n, t: (0, 0)),
            pl.BlockSpec((Bn, Pd), lambda n, t: (0, 0)),
            pl.BlockSpec((Bn, 1), lambda n, t: (0, 0)),
            pl.BlockSpec((K, Bn), lambda n, t: (0, 0)),
            pl.BlockSpec((K, 1), lambda n, t: (0, 0)),
            pl.BlockSpec((H, h), lambda n, t: (0, 0)),
            pl.BlockSpec((w, W), lambda n, t: (0, 0)),
        ],
        out_specs=pl.BlockSpec(memory_space=pltpu.MemorySpace.HBM),
        out_shape=jax.ShapeDtypeStruct((N, K, H, W), jnp.float32),
        scratch_shapes=[
            pltpu.VMEM((C * H, W), jnp.float32),
            pltpu.VMEM((C64 + 8, P), jnp.float32),
            pltpu.VMEM((E, P), jnp.bfloat16),
            pltpu.VMEM((Bn, P), jnp.bfloat16),
            pltpu.VMEM((K, P), jnp.bfloat16),
            pltpu.VMEM((K, h, W), jnp.bfloat16),
            pltpu.VMEM((12, H, W), jnp.float32),
            pltpu.SemaphoreType.DMA((12,)),
        ],
        compiler_params=pltpu.CompilerParams(
            dimension_semantics=("arbitrary", "arbitrary")),
    )(x, sall, wselb, poolwt, poolbt, wmt, bwpt, bbt, cwt, cbt, Rh, Rw_T)


# speedup vs baseline: 1.6993x; 1.0007x over previous
import functools

import numpy as np
import jax
import jax.numpy as jnp
from jax.experimental import pallas as pl
from jax.experimental.pallas import tpu as pltpu


def _interp_matrix_np(out_size, in_size):
    if in_size == 1:
        return np.ones((out_size, 1), np.float32)
    if out_size == 1:
        src = np.zeros((1,), np.float64)
    else:
        src = np.arange(out_size, dtype=np.float64) * (in_size - 1) / (out_size - 1)
    i0 = np.clip(np.floor(src).astype(np.int64), 0, in_size - 1)
    i1 = np.minimum(i0 + 1, in_size - 1)
    frac = (src - i0).astype(np.float32)
    R = np.zeros((out_size, in_size), np.float32)
    R[np.arange(out_size), i0] += 1.0 - frac
    R[np.arange(out_size), i1] += frac
    return R


def _mega_kernel(x_ref, sall_ref, wselb_ref, poolwt_ref, poolbt_ref,
                 wmt_ref, bwpt_ref, bbt_ref, cwt_ref, cbt_ref, rh_ref, rwt_ref,
                 o_ref, selall_ref, selcat_ref, ft_ref, feats_ref, lg_ref,
                 v3_ref, obuf_ref, osem, *, dims, inv_p, nk):
    C, h, w, H, W = dims
    C64 = C * 64
    n = pl.program_id(0)
    t = pl.program_id(1)
    K = cwt_ref.shape[0]
    D = obuf_ref.shape[0]
    sid = n * K + t
    slot = jax.lax.rem(sid, D)

    @pl.when(t == 0)
    def _head():
        E = ft_ref.shape[0]
        K = cwt_ref.shape[0]
        P = h * w

        x2 = x_ref[0].reshape(C * H, W).astype(jnp.bfloat16)
        selall_ref[...] = jnp.dot(x2, sall_ref[...],
                                  preferred_element_type=jnp.float32
                                  ).astype(jnp.bfloat16)

        selcat_ref[pl.ds(C64, 16), :] = jnp.ones((16, P), jnp.bfloat16)

        def band(i4, _):
            for c in range(C):
                b = selall_ref[pl.ds(c * H + 32 * i4, 32), :]
                for dx2 in range(4):
                    piece = jnp.concatenate(
                        [jnp.concatenate(
                            [b[8 * s:8 * (s + 1), w * dx:w * (dx + 1)]
                             for s in range(4)], axis=1)
                         for dx in (2 * dx2, 2 * dx2 + 1)], axis=0)
                    selcat_ref[pl.ds(c * 64 + dx2 * 16, 16),
                               pl.ds(4 * w * i4, 4 * w)] = piece
            return 0

        jax.lax.fori_loop(0, h // 4, band, 0)

        f = jnp.maximum(jnp.dot(wselb_ref[...], selcat_ref[...],
                                preferred_element_type=jnp.float32), 0.0)
        ft_ref[...] = f.astype(jnp.bfloat16)

        mean = jnp.sum(f, axis=1, keepdims=True) * inv_p
        hdn = jnp.dot(poolwt_ref[...], mean,
                      preferred_element_type=jnp.float32) + poolbt_ref[...]
        hdn = jnp.maximum(hdn, 0.0)
        pc = jnp.dot(bwpt_ref[...], hdn,
                     preferred_element_type=jnp.float32) + bbt_ref[...]

        feats = jnp.dot(wmt_ref[...], ft_ref[...],
                        preferred_element_type=jnp.float32)
        feats_ref[...] = jnp.maximum(feats + pc, 0.0).astype(jnp.bfloat16)
        lg_ref[...] = (jnp.dot(cwt_ref[...], feats_ref[...],
                               preferred_element_type=jnp.float32)
                       + cbt_ref[...]).astype(jnp.bfloat16)

        def colpass(i16, _):
            lgp = lg_ref[:, pl.ds(16 * w * i16, 16 * w)]
            vs = [jnp.dot(lgp[:, w * s:w * (s + 1)], rwt_ref[...],
                          preferred_element_type=jnp.float32)
                  .astype(jnp.bfloat16).reshape(K, 1, W)
                  for s in range(16)]
            v3_ref[:, pl.ds(16 * i16, 16), :] = jnp.concatenate(vs, axis=1)
            return 0

        jax.lax.fori_loop(0, h // 16, colpass, 0)

    @pl.when(sid >= D)
    def _reclaim():
        pltpu.make_async_copy(obuf_ref.at[slot], o_ref.at[0, 0],
                              osem.at[slot]).wait()

    vk = v3_ref[t]
    obuf_ref[slot] = jnp.dot(rh_ref[...], vk,
                             preferred_element_type=jnp.float32)
    pltpu.make_async_copy(obuf_ref.at[slot], o_ref.at[n, t],
                          osem.at[slot]).start()

    @pl.when(sid == nk - 1)
    def _drain():
        for d in range(D):
            pltpu.make_async_copy(obuf_ref.at[d], o_ref.at[0, 0],
                                  osem.at[d]).wait()


def kernel(patch_w, patch_b, pool_w, pool_b, bott_w_main, bott_w_pool,
           bott_b, cls_w, cls_b, x):
    N, C, H, W = x.shape
    h, w = H // 8, W // 8
    P = h * w
    C64 = C * 64
    E = patch_w.shape[1]
    Pd = pool_w.shape[1]
    Bn = bott_w_main.shape[1]
    K = cls_w.shape[1]

    sall_np = np.zeros((W, W), np.float32)
    for dx in range(8):
        sall_np[8 * np.arange(w) + dx, dx * w + np.arange(w)] = 1.0
    sall = jnp.asarray(sall_np, jnp.bfloat16)

    wsel = (patch_w.reshape(C, 8, 8, E).transpose(0, 2, 1, 3)
            .reshape(C64, E).T)
    wselb = jnp.concatenate(
        [wsel, patch_b.T.astype(jnp.bfloat16),
         jnp.zeros((E, 15), jnp.bfloat16)], axis=1)

    poolwt = pool_w.T
    poolbt = pool_b.T
    wmt = bott_w_main.T
    bwpt = bott_w_pool.T
    bbt = bott_b.T
    cwt = cls_w.T
    cbt = cls_b.T

    Rh = jnp.asarray(_interp_matrix_np(H, h), jnp.bfloat16)
    Rw_T = jnp.asarray(_interp_matrix_np(W, w).T,
                       jnp.bfloat16)

    return pl.pallas_call(
        functools.partial(_mega_kernel, dims=(C, h, w, H, W),
                          inv_p=1.0 / float(P), nk=N * K),
        grid=(N, K),
        in_specs=[
            pl.BlockSpec((1, C, H, W), lambda n, t: (n, 0, 0, 0)),
            pl.BlockSpec((W, W), lambda n, t: (0, 0)),
            pl.BlockSpec((E, C64 + 16), lambda n, t: (0, 0)),
            pl.BlockSpec((Pd, E), lambda n, t: (0, 0)),
            pl.BlockSpec((Pd, 1), lambda n, t: (0, 0)),
            pl.BlockSpec((Bn, E), lambda n, t: (0, 0)),
            pl.BlockSpec((Bn, Pd), lambda n, t: (0, 0)),
            pl.BlockSpec((Bn, 1), lambda n, t: (0, 0)),
            pl.BlockSpec((K, Bn), lambda n, t: (0, 0)),
            pl.BlockSpec((K, 1), lambda n, t: (0, 0)),
            pl.BlockSpec((H, h), lambda n, t: (0, 0)),
            pl.BlockSpec((w, W), lambda n, t: (0, 0)),
        ],
        out_specs=pl.BlockSpec(memory_space=pltpu.MemorySpace.HBM),
        out_shape=jax.ShapeDtypeStruct((N, K, H, W), jnp.float32),
        scratch_shapes=[
            pltpu.VMEM((C * H, W), jnp.bfloat16),
            pltpu.VMEM((C64 + 16, P), jnp.bfloat16),
            pltpu.VMEM((E, P), jnp.bfloat16),
            pltpu.VMEM((Bn, P), jnp.bfloat16),
            pltpu.VMEM((K, P), jnp.bfloat16),
            pltpu.VMEM((K, h, W), jnp.bfloat16),
            pltpu.VMEM((16, H, W), jnp.float32),
            pltpu.SemaphoreType.DMA((16,)),
        ],
        compiler_params=pltpu.CompilerParams(
            dimension_semantics=("arbitrary", "arbitrary")),
    )(x, sall, wselb, poolwt, poolbt, wmt, bwpt, bbt, cwt, cbt, Rh, Rw_T)
